# baseline (speedup 1.0000x reference)
import jax
import jax.numpy as jnp
from jax import lax
from jax.experimental import pallas as pl
from jax.experimental.pallas import tpu as pltpu
from jax.experimental.pallas import tpu_sc as plsc

RES = 512
FEAT = 32
NP = 3
NC = 2
NS = 16
NW = NC * NS
L = 16
CH = 128
POFF = RES * RES


TW = 1024


def _tr_body(m2, tab, buf, bufT):
    rows_total = NP * RES * RES
    per_w = rows_total // NW
    wid = lax.axis_index("s") * NC + lax.axis_index("c")
    row0 = wid * per_w
    lanes = jnp.arange(L, dtype=jnp.int32)

    def chunk(t, carry):
        rowbase = row0 + t * TW
        p = rowbase // (RES * RES)
        cb = rowbase % (RES * RES)
        pltpu.sync_copy(m2.at[p, :, pl.ds(cb, TW)], buf)

        def k_loop(k, c2):
            rvec = k * L + lanes
            for f in range(FEAT):
                vec = buf[f, pl.ds(k * L, L)]
                plsc.store_scatter(bufT, [rvec, jnp.full((L,), f, jnp.int32)],
                                   vec)
            return c2

        lax.fori_loop(0, TW // L, k_loop, 0)
        pltpu.sync_copy(bufT, tab.at[pl.ds(rowbase, TW)])
        return carry

    lax.fori_loop(0, per_w // TW, chunk, 0)


def _make_table(mat):
    m2 = mat.reshape(NP, FEAT, RES * RES)
    mesh = plsc.VectorSubcoreMesh(core_axis_name="c", subcore_axis_name="s",
                                  num_cores=NC, num_subcores=NS)
    f = pl.kernel(
        _tr_body,
        out_type=jax.ShapeDtypeStruct((NP * RES * RES, FEAT), jnp.float32),
        mesh=mesh,
        compiler_params=pltpu.CompilerParams(use_tc_tiling_on_sc=False,
                                             needs_layout_passes=False),
        scratch_types=[
            pltpu.VMEM((FEAT, TW), jnp.float32),
            pltpu.VMEM((TW, FEAT), jnp.float32),
        ],
    )
    return f(m2)


def _tri_body(t0, t1, t2, x0, x1, x2, out, xc, idxs, wts, rows, outv,
              sg0, sg1, sx0, sx1, osem):
    B = out.shape[0]
    pw = B // NW
    nch = pw // CH
    wid = lax.axis_index("s") * NC + lax.axis_index("c")
    base0 = wid * pw
    tabs = (t0, t1, t2)
    xs = (x0, x1, x2)
    gsems = (sg0, sg1)
    xsems = (sx0, sx1)

    def x_descs(g, buf):
        base = base0 + g * CH
        return [pltpu.make_async_copy(xs[p].at[pl.ds(base, CH)],
                                      xc.at[buf, p], xsems[buf])
                for p in range(NP)]

    def compute_chunk(g, buf):

        def grp(g2):
            s = g2 * L
            u0 = xc[buf, 0, pl.ds(s, L)]
            u1 = xc[buf, 1, pl.ds(s, L)]
            u2 = xc[buf, 2, pl.ds(s, L)]
            for p, (ua, ub) in enumerate(((u0, u1), (u1, u2), (u2, u0))):
                uu = ua * (RES - 1.0)
                vv = ub * (RES - 1.0)
                iu = jnp.clip(uu.astype(jnp.int32), 0, RES - 1)
                jv = jnp.clip(vv.astype(jnp.int32), 0, RES - 1)
                wi = uu - iu.astype(jnp.float32)
                wj = vv - jv.astype(jnp.float32)
                di = jnp.minimum(iu + 1, RES - 1) - iu
                dj = jnp.minimum(jv + 1, RES - 1) - jv
                b00 = iu * RES + jv
                idxs[buf, p, pl.ds(0 * CH + s, L)] = b00
                idxs[buf, p, pl.ds(1 * CH + s, L)] = b00 + dj
                idxs[buf, p, pl.ds(2 * CH + s, L)] = b00 + di * RES
                idxs[buf, p, pl.ds(3 * CH + s, L)] = b00 + di * RES + dj
                oi = 1.0 - wi
                oj = 1.0 - wj
                r = 4 * p
                wts[buf, r + 0, pl.ds(s, L)] = oi * oj
                wts[buf, r + 1, pl.ds(s, L)] = oi * wj
                wts[buf, r + 2, pl.ds(s, L)] = wi * oj
                wts[buf, r + 3, pl.ds(s, L)] = wi * wj

        plsc.parallel_loop(0, CH // L)(grp)

    def gather_descs(buf):
        return [pltpu.make_async_copy(tabs[p].at[idxs.at[buf, p]],
                                      rows.at[buf, p], gsems[buf])
                for p in range(NP)]

    def out_desc(g):
        base = base0 + g * CH
        return pltpu.make_async_copy(outv, out.at[pl.ds(base, CH)], osem)

    def blend(g, buf):
        @pl.when(g >= 1)
        def _():
            out_desc(g).wait()

        def grp(g2):
            s = g2 * L
            wv = [wts[buf, k, pl.ds(s, L)] for k in range(4 * NP)]
            for i in range(L):
                c = s + i
                li = jnp.full((L,), i, dtype=jnp.int32)
                accs = []
                for p in range(NP):
                    r = 4 * p
                    w00 = wv[r + 0][li]
                    w01 = wv[r + 1][li]
                    w10 = wv[r + 2][li]
                    w11 = wv[r + 3][li]
                    for h in range(FEAT // L):
                        sl = pl.ds(h * L, L)
                        accs.append((rows[buf, p, 0 * CH + c, sl] * w00
                                     + rows[buf, p, 1 * CH + c, sl] * w01)
                                    + (rows[buf, p, 2 * CH + c, sl] * w10
                                       + rows[buf, p, 3 * CH + c, sl] * w11))
                for k, acc in enumerate(accs):
                    outv[c, pl.ds(k * L, L)] = acc

        plsc.parallel_loop(0, CH // L)(grp)
        out_desc(g).start()

    for _d in x_descs(0, 0):
        _d.start()
    for _d in x_descs(0, 0):
        _d.wait()
    compute_chunk(0, 0)
    for _d in gather_descs(0):
        _d.start()
    for _d in x_descs(1, 1):
        _d.start()

    def body(i, carry):
        g0 = 2 * i
        g1 = 2 * i + 1
        g2w = (2 * i + 2) & (nch - 1)
        g3w = (2 * i + 3) & (nch - 1)
        for d in x_descs(g1, 1):
            d.wait()
        compute_chunk(g1, 1)
        for d in gather_descs(1):
            d.start()
        for d in x_descs(g2w, 0):
            d.start()
        for d in gather_descs(0):
            d.wait()
        blend(g0, 0)
        for d in x_descs(g2w, 0):
            d.wait()
        compute_chunk(g2w, 0)
        for d in gather_descs(0):
            d.start()
        for d in x_descs(g3w, 1):
            d.start()
        for d in gather_descs(1):
            d.wait()
        blend(g1, 1)
        return carry

    lax.fori_loop(0, nch // 2, body, 0)
    for _d in gather_descs(0):
        _d.wait()
    for _d in x_descs(1, 1):
        _d.wait()
    out_desc(0).wait()


def kernel(x, mat):
    B = x.shape[0]
    assert B % (NW * CH) == 0
    tab = jnp.transpose(mat, (0, 2, 3, 1)).reshape(NP, RES * RES, FEAT)
    mesh = plsc.VectorSubcoreMesh(core_axis_name="c", subcore_axis_name="s",
                                  num_cores=NC, num_subcores=NS)
    f = pl.kernel(
        _tri_body,
        out_type=jax.ShapeDtypeStruct((B, NP * FEAT), jnp.float32),
        mesh=mesh,
        compiler_params=pltpu.CompilerParams(use_tc_tiling_on_sc=False),
        scratch_types=[
            pltpu.VMEM((2, NP, CH), jnp.float32),
            pltpu.VMEM((2, NP, 4 * CH), jnp.int32),
            pltpu.VMEM((2, 12, CH), jnp.float32),
            pltpu.VMEM((2, NP, 4 * CH, FEAT), jnp.float32),
            pltpu.VMEM((CH, NP * FEAT), jnp.float32),
            pltpu.SemaphoreType.DMA,
            pltpu.SemaphoreType.DMA,
            pltpu.SemaphoreType.DMA,
            pltpu.SemaphoreType.DMA,
            pltpu.SemaphoreType.DMA,
        ],
    )
    return f(tab[0], tab[1], tab[2], x[:, 0], x[:, 1], x[:, 2])


# speedup vs baseline: 1.0174x; 1.0174x over previous
import jax
import jax.numpy as jnp
from jax import lax
from jax.experimental import pallas as pl
from jax.experimental.pallas import tpu as pltpu
from jax.experimental.pallas import tpu_sc as plsc

RES = 512
FEAT = 32
NP = 3
NC = 2
NS = 16
NW = NC * NS
L = 16
CH = 128
POFF = RES * RES


TW = 1024


def _tr_body(m2, tab, buf, bufT):
    rows_total = NP * RES * RES
    per_w = rows_total // NW
    wid = lax.axis_index("s") * NC + lax.axis_index("c")
    row0 = wid * per_w
    lanes = jnp.arange(L, dtype=jnp.int32)

    def chunk(t, carry):
        rowbase = row0 + t * TW
        p = rowbase // (RES * RES)
        cb = rowbase % (RES * RES)
        pltpu.sync_copy(m2.at[p, :, pl.ds(cb, TW)], buf)

        def k_loop(k, c2):
            rvec = k * L + lanes
            for f in range(FEAT):
                vec = buf[f, pl.ds(k * L, L)]
                plsc.store_scatter(bufT, [rvec, jnp.full((L,), f, jnp.int32)],
                                   vec)
            return c2

        lax.fori_loop(0, TW // L, k_loop, 0)
        pltpu.sync_copy(bufT, tab.at[pl.ds(rowbase, TW)])
        return carry

    lax.fori_loop(0, per_w // TW, chunk, 0)


def _make_table(mat):
    m2 = mat.reshape(NP, FEAT, RES * RES)
    mesh = plsc.VectorSubcoreMesh(core_axis_name="c", subcore_axis_name="s",
                                  num_cores=NC, num_subcores=NS)
    f = pl.kernel(
        _tr_body,
        out_type=jax.ShapeDtypeStruct((NP * RES * RES, FEAT), jnp.float32),
        mesh=mesh,
        compiler_params=pltpu.CompilerParams(use_tc_tiling_on_sc=False,
                                             needs_layout_passes=False),
        scratch_types=[
            pltpu.VMEM((FEAT, TW), jnp.float32),
            pltpu.VMEM((TW, FEAT), jnp.float32),
        ],
    )
    return f(m2)


def _tri_body(t0, t1, t2, x0, x1, x2, out, xc, idxs, wts, rows, outv,
              sg0, sg1, sx0, sx1, osem):
    B = out.shape[0]
    pw = B // NW
    nch = pw // CH
    wid = lax.axis_index("s") * NC + lax.axis_index("c")
    base0 = wid * pw
    tabs = (t0, t1, t2)
    xs = (x0, x1, x2)
    gsems = (sg0, sg1)
    xsems = (sx0, sx1)

    def x_descs(g, buf):
        base = base0 + g * CH
        return [pltpu.make_async_copy(xs[p].at[pl.ds(base, CH)],
                                      xc.at[buf, p], xsems[buf])
                for p in range(NP)]

    def compute_chunk(g, buf):

        def grp(g2):
            s = g2 * L
            u0 = xc[buf, 0, pl.ds(s, L)]
            u1 = xc[buf, 1, pl.ds(s, L)]
            u2 = xc[buf, 2, pl.ds(s, L)]
            for p, (ua, ub) in enumerate(((u0, u1), (u1, u2), (u2, u0))):
                uu = ua * (RES - 1.0)
                vv = ub * (RES - 1.0)
                iu = jnp.clip(uu.astype(jnp.int32), 0, RES - 1)
                jv = jnp.clip(vv.astype(jnp.int32), 0, RES - 1)
                wi = uu - iu.astype(jnp.float32)
                wj = vv - jv.astype(jnp.float32)
                di = jnp.minimum(iu + 1, RES - 1) - iu
                dj = jnp.minimum(jv + 1, RES - 1) - jv
                b00 = iu * RES + jv
                idxs[buf, p, pl.ds(0 * CH + s, L)] = b00
                idxs[buf, p, pl.ds(1 * CH + s, L)] = b00 + dj
                idxs[buf, p, pl.ds(2 * CH + s, L)] = b00 + di * RES
                idxs[buf, p, pl.ds(3 * CH + s, L)] = b00 + di * RES + dj
                oi = 1.0 - wi
                oj = 1.0 - wj
                r = 4 * p
                wts[buf, r + 0, pl.ds(s, L)] = oi * oj
                wts[buf, r + 1, pl.ds(s, L)] = oi * wj
                wts[buf, r + 2, pl.ds(s, L)] = wi * oj
                wts[buf, r + 3, pl.ds(s, L)] = wi * wj

        plsc.parallel_loop(0, CH // L)(grp)

    def gather_descs(buf):
        return [pltpu.make_async_copy(tabs[p].at[idxs.at[buf, p]],
                                      rows.at[buf, p], gsems[buf])
                for p in range(NP)]

    def out_desc(g):
        base = base0 + g * CH
        return pltpu.make_async_copy(outv, out.at[pl.ds(base, CH)], osem)

    def blend(g, buf):
        @pl.when(g >= 1)
        def _():
            out_desc(g).wait()

        def grp(g2):
            s = g2 * L
            wv = [wts[buf, k, pl.ds(s, L)] for k in range(4 * NP)]
            for i in range(L):
                c = s + i
                li = jnp.full((L,), i, dtype=jnp.int32)
                accs = []
                for p in range(NP):
                    r = 4 * p
                    w00 = wv[r + 0][li]
                    w01 = wv[r + 1][li]
                    w10 = wv[r + 2][li]
                    w11 = wv[r + 3][li]
                    for h in range(FEAT // L):
                        sl = pl.ds(h * L, L)
                        accs.append(rows[buf, p, 0 * CH + c, sl] * w00
                                    + rows[buf, p, 1 * CH + c, sl] * w01
                                    + rows[buf, p, 2 * CH + c, sl] * w10
                                    + rows[buf, p, 3 * CH + c, sl] * w11)
                for k, acc in enumerate(accs):
                    outv[c, pl.ds(k * L, L)] = acc

        plsc.parallel_loop(0, CH // L)(grp)
        out_desc(g).start()

    for _d in x_descs(0, 0):
        _d.start()
    for _d in x_descs(0, 0):
        _d.wait()
    compute_chunk(0, 0)
    for _d in gather_descs(0):
        _d.start()
    for _d in x_descs(1, 1):
        _d.start()

    def body(i, carry):
        g0 = 2 * i
        g1 = 2 * i + 1
        g2w = (2 * i + 2) & (nch - 1)
        g3w = (2 * i + 3) & (nch - 1)
        for d in x_descs(g1, 1):
            d.wait()
        compute_chunk(g1, 1)
        for d in gather_descs(1):
            d.start()
        for d in x_descs(g2w, 0):
            d.start()
        for d in gather_descs(0):
            d.wait()
        blend(g0, 0)
        for d in x_descs(g2w, 0):
            d.wait()
        compute_chunk(g2w, 0)
        for d in gather_descs(0):
            d.start()
        for d in x_descs(g3w, 1):
            d.start()
        for d in gather_descs(1):
            d.wait()
        blend(g1, 1)
        return carry

    lax.fori_loop(0, nch // 2, body, 0)
    for _d in gather_descs(0):
        _d.wait()
    for _d in x_descs(1, 1):
        _d.wait()
    out_desc(0).wait()


def kernel(x, mat):
    B = x.shape[0]
    assert B % (NW * CH) == 0
    tab = jnp.transpose(mat, (0, 2, 3, 1)).reshape(NP, RES * RES, FEAT)
    mesh = plsc.VectorSubcoreMesh(core_axis_name="c", subcore_axis_name="s",
                                  num_cores=NC, num_subcores=NS)
    f = pl.kernel(
        _tri_body,
        out_type=jax.ShapeDtypeStruct((B, NP * FEAT), jnp.float32),
        mesh=mesh,
        compiler_params=pltpu.CompilerParams(use_tc_tiling_on_sc=False),
        scratch_types=[
            pltpu.VMEM((2, NP, CH), jnp.float32),
            pltpu.VMEM((2, NP, 4 * CH), jnp.int32),
            pltpu.VMEM((2, 12, CH), jnp.float32),
            pltpu.VMEM((2, NP, 4 * CH, FEAT), jnp.float32),
            pltpu.VMEM((CH, NP * FEAT), jnp.float32),
            pltpu.SemaphoreType.DMA,
            pltpu.SemaphoreType.DMA,
            pltpu.SemaphoreType.DMA,
            pltpu.SemaphoreType.DMA,
            pltpu.SemaphoreType.DMA,
        ],
    )
    return f(tab[0], tab[1], tab[2], x[:, 0], x[:, 1], x[:, 2])
